# baseline (speedup 1.0000x reference)
import functools

import jax
import jax.numpy as jnp
from jax import lax
from jax.experimental import pallas as pl
from jax.experimental.pallas import tpu as pltpu
from jax.experimental.pallas import tpu_sc as plsc

DELTA_V = 0.5
DELTA_D = 1.5
P_VAR = 1.0
P_DIST = 1.0
P_REG = 0.001
NUM_CLASSES = 24
NUM_SEM = 13

_NW = 32
_TPS = 4
_QTR = 1024
_CH = 256


def _sc_seg_body(pred_hbm, lbl_hbm, out_hbm, lbl_v, buf0, buf1, seg_v, sem0,
                 sem1, *, s_base):
    wid = lax.axis_index("s") * 2 + lax.axis_index("c")
    s_loc = wid // _TPS
    h = wid % _TPS
    s = s_base + s_loc
    base_pt = h * _QTR

    zero = jnp.zeros((16,), jnp.float32)
    for row in range(NUM_CLASSES):
        for k in range(8):
            seg_v[row, pl.ds(k * 16, 16)] = zero

    pltpu.sync_copy(lbl_hbm.at[s, 0, pl.ds(base_pt, _QTR)], lbl_v)

    lane = lax.iota(jnp.int32, 16)
    bufs = (buf0, buf1)
    sems = (sem0, sem1)
    nch = _QTR // _CH

    def start(c):
        return pltpu.async_copy(
            pred_hbm.at[s, pl.ds(base_pt + c * _CH, _CH), :],
            bufs[c % 2], sems[c % 2])

    copies = {0: start(0)}
    for c in range(nch):
        if c + 1 < nch:
            copies[c + 1] = start(c + 1)
        copies[c].wait()
        buf = bufs[c % 2]

        @plsc.parallel_loop(0, _CH // 16, step=1)
        def blk_body(blk, c=c, buf=buf):
            p0 = blk * 16
            lbl_vec = lbl_v[pl.ds(c * _CH + p0, 16)]

            def load(j):
                vs = [buf[p0 + j, pl.ds(k * 16, 16)] for k in range(8)]
                row = jnp.full((16,), lbl_vec[j], jnp.int32)
                return vs, row

            vecs, row = load(0)
            for j in range(1, 16):
                nvecs, nrow = load(j)
                for k in range(8):
                    plsc.addupdate_scatter(
                        seg_v, [row, lane + k * 16], vecs[k])
                vecs, row = nvecs, nrow
            for k in range(8):
                plsc.addupdate_scatter(seg_v, [row, lane + k * 16], vecs[k])

    pltpu.sync_copy(seg_v, out_hbm.at[s_loc, h])


def _sc_segment_sums(pred_ins, ins_lbl, s_base):
    B, n, D = pred_ins.shape
    nw_samples = _NW // _TPS
    mesh = plsc.VectorSubcoreMesh(core_axis_name="c", subcore_axis_name="s")
    f = pl.kernel(
        functools.partial(_sc_seg_body, s_base=s_base),
        out_type=jax.ShapeDtypeStruct(
            (nw_samples, _TPS, NUM_CLASSES, 128), jnp.float32),
        mesh=mesh,
        scratch_types=[
            pltpu.VMEM((_QTR,), jnp.int32),
            pltpu.VMEM((_CH, D), jnp.float32),
            pltpu.VMEM((_CH, D), jnp.float32),
            pltpu.VMEM((NUM_CLASSES, 128), jnp.float32),
            pltpu.SemaphoreType.DMA,
            pltpu.SemaphoreType.DMA,
        ],
        compiler_params=pltpu.CompilerParams(needs_layout_passes=False),
    )
    return f(pred_ins, ins_lbl)


def _disc_body(pred_ins_ref, ins_lbl_ref, seg_ref, out_ref, *, nsamp):
    i = pl.program_id(0)
    n = pred_ins_ref.shape[1]
    M = NUM_CLASSES

    pred = pred_ins_ref[0]
    lbl_row = ins_lbl_ref[0]
    seg = (seg_ref[0, 0] + seg_ref[0, 1]) + (seg_ref[0, 2] + seg_ref[0, 3])

    cls_iota = jax.lax.broadcasted_iota(jnp.int32, (M, n), 0)
    oh_t = (cls_iota == lbl_row).astype(jnp.float32)
    counts_col = jnp.sum(oh_t, axis=1, keepdims=True)

    present_col = counts_col > 0.0
    presentf_col = present_col.astype(jnp.float32)
    Kf = jnp.sum(presentf_col)
    cf_safe = jnp.where(present_col, counts_col, 1.0)
    mu = jnp.where(present_col, seg / cf_safe, 0.0)

    ones_col = jnp.ones((128, 1), dtype=jnp.float32)

    mu_exp = jax.lax.dot_general(
        oh_t, mu, (((0,), (0,)), ((), ())),
        preferred_element_type=jnp.float32)
    ad = jnp.abs(pred - mu_exp)
    dist = jnp.dot(ad, ones_col, preferred_element_type=jnp.float32)
    r = jnp.square(jnp.maximum(dist - DELTA_V, 0.0))
    segr = jnp.dot(oh_t, r, preferred_element_type=jnp.float32)
    l_var = jnp.sum(segr / cf_safe) / Kf

    diff = jnp.abs(mu[:, None, :] - mu[None, :, :]).reshape(M * M, 128)
    n1 = jnp.dot(diff, ones_col,
                 preferred_element_type=jnp.float32).reshape(M, M)
    mn = jnp.square(jnp.maximum(2.0 * DELTA_D - n1, 0.0))
    row_i = jax.lax.broadcasted_iota(jnp.int32, (M, M), 0)
    col_i = jax.lax.broadcasted_iota(jnp.int32, (M, M), 1)
    off_diag = (row_i != col_i).astype(jnp.float32)
    pair_mask = presentf_col * presentf_col.reshape(1, M) * off_diag
    denom = jnp.where(Kf > 1.0, Kf * (Kf - 1.0), 1.0)
    l_dist = jnp.where(Kf > 1.0, jnp.sum(mn * pair_mask) / denom, 0.0)

    l_reg = jnp.sum(jnp.abs(mu)) / Kf
    disc = P_VAR * l_var + P_DIST * l_dist + P_REG * l_reg

    contrib = disc / nsamp
    prev = jnp.where(i == 0, jnp.zeros((1, 1), jnp.float32), out_ref[...])
    out_ref[...] = prev + contrib


def _ce_body(pred_sem_ref, sem_lbl_ref, out_ref):
    i = pl.program_id(0)
    n = pred_sem_ref.shape[2]

    x = pred_sem_ref[0]
    sem_row = sem_lbl_ref[0]
    m = jnp.max(x, axis=0, keepdims=True)
    lse = jnp.log(jnp.sum(jnp.exp(x - m), axis=0, keepdims=True)) + m
    sem_iota = jax.lax.broadcasted_iota(jnp.int32, (NUM_SEM, n), 0)
    oh_sem_t = (sem_iota == sem_row).astype(jnp.float32)
    xl = jnp.sum(x * oh_sem_t, axis=0, keepdims=True)
    nll_sum = jnp.sum(lse - xl)

    B = pl.num_programs(0)
    contrib = nll_sum / (B * n)
    prev = jnp.where(i == 0, jnp.zeros((1, 1), jnp.float32), out_ref[...])
    out_ref[...] = prev + contrib


def _disc_call(pred_bf, ins_lbl, seg_parts, s_base, nsamp, nwave):
    n, D = pred_bf.shape[1], pred_bf.shape[2]
    return pl.pallas_call(
        functools.partial(_disc_body, nsamp=nsamp),
        grid=(nwave,),
        in_specs=[
            pl.BlockSpec((1, n, D), lambda i: (i + s_base, 0, 0)),
            pl.BlockSpec((1, 1, n), lambda i: (i + s_base, 0, 0)),
            pl.BlockSpec((1, _TPS, NUM_CLASSES, 128), lambda i: (i, 0, 0, 0)),
        ],
        out_specs=pl.BlockSpec((1, 1), lambda i: (0, 0)),
        out_shape=jax.ShapeDtypeStruct((1, 1), jnp.float32),
    )(pred_bf, ins_lbl, seg_parts)


def kernel(pred_ins, pred_sem, true_ins, true_sem):
    B, n, D = pred_ins.shape
    nwave = _NW // _TPS
    sem_t = jnp.transpose(pred_sem, (0, 2, 1))
    ins_lbl = true_ins.reshape(B, 1, n)
    sem_lbl = true_sem.reshape(B, 1, n)

    seg_a = _sc_segment_sums(pred_ins, ins_lbl, 0)
    seg_b = _sc_segment_sums(pred_ins, ins_lbl, nwave)

    ce = pl.pallas_call(
        _ce_body,
        grid=(B,),
        in_specs=[
            pl.BlockSpec((1, NUM_SEM, n), lambda i: (i, 0, 0)),
            pl.BlockSpec((1, 1, n), lambda i: (i, 0, 0)),
        ],
        out_specs=pl.BlockSpec((1, 1), lambda i: (0, 0)),
        out_shape=jax.ShapeDtypeStruct((1, 1), jnp.float32),
    )(sem_t, sem_lbl)

    disc_a = _disc_call(pred_ins, ins_lbl, seg_a, 0, B, nwave)
    disc_b = _disc_call(pred_ins, ins_lbl, seg_b, nwave, B, nwave)

    return (pred_sem, ce[0, 0] + disc_a[0, 0] + disc_b[0, 0])


# speedup vs baseline: 1.0593x; 1.0593x over previous
import functools

import jax
import jax.numpy as jnp
from jax import lax
from jax.experimental import pallas as pl
from jax.experimental.pallas import tpu as pltpu
from jax.experimental.pallas import tpu_sc as plsc

DELTA_V = 0.5
DELTA_D = 1.5
P_VAR = 1.0
P_DIST = 1.0
P_REG = 0.001
NUM_CLASSES = 24
NUM_SEM = 13

_NW = 32
_TPS = 4
_QTR = 1024
_CH = 256


def _sc_seg_body(pred_hbm, lbl_hbm, out_hbm, lbl_v, buf0, buf1, seg_v, sem0,
                 sem1, *, s_base):
    wid = lax.axis_index("s") * 2 + lax.axis_index("c")
    s_loc = wid // _TPS
    h = wid % _TPS
    s = s_base + s_loc
    base_pt = h * _QTR

    zero = jnp.zeros((16,), jnp.float32)
    for row in range(NUM_CLASSES):
        for k in range(8):
            seg_v[row, pl.ds(k * 16, 16)] = zero

    pltpu.sync_copy(lbl_hbm.at[s, 0, pl.ds(base_pt, _QTR)], lbl_v)

    lane = lax.iota(jnp.int32, 16)
    bufs = (buf0, buf1)
    sems = (sem0, sem1)
    nch = _QTR // _CH

    def start(c):
        return pltpu.async_copy(
            pred_hbm.at[s, pl.ds(base_pt + c * _CH, _CH), :],
            bufs[c % 2], sems[c % 2])

    copies = {0: start(0)}
    for c in range(nch):
        if c + 1 < nch:
            copies[c + 1] = start(c + 1)
        copies[c].wait()
        buf = bufs[c % 2]

        def blk_body(blk, cc, c=c, buf=buf):
            p0 = blk * 16
            lbl_vec = lbl_v[pl.ds(c * _CH + p0, 16)]

            def load(j):
                vs = [buf[p0 + j, pl.ds(k * 16, 16)] for k in range(8)]
                row = jnp.full((16,), lbl_vec[j], jnp.int32)
                return vs, row

            vecs, row = load(0)
            for j in range(1, 16):
                nvecs, nrow = load(j)
                for k in range(8):
                    plsc.addupdate_scatter(
                        seg_v, [row, lane + k * 16], vecs[k])
                vecs, row = nvecs, nrow
            for k in range(8):
                plsc.addupdate_scatter(seg_v, [row, lane + k * 16], vecs[k])
            return cc

        lax.fori_loop(0, _CH // 16, blk_body, 0)

    pltpu.sync_copy(seg_v, out_hbm.at[s_loc, h])


def _sc_segment_sums(pred_ins, ins_lbl, s_base):
    B, n, D = pred_ins.shape
    nw_samples = _NW // _TPS
    mesh = plsc.VectorSubcoreMesh(core_axis_name="c", subcore_axis_name="s")
    f = pl.kernel(
        functools.partial(_sc_seg_body, s_base=s_base),
        out_type=jax.ShapeDtypeStruct(
            (nw_samples, _TPS, NUM_CLASSES, 128), jnp.float32),
        mesh=mesh,
        scratch_types=[
            pltpu.VMEM((_QTR,), jnp.int32),
            pltpu.VMEM((_CH, D), jnp.float32),
            pltpu.VMEM((_CH, D), jnp.float32),
            pltpu.VMEM((NUM_CLASSES, 128), jnp.float32),
            pltpu.SemaphoreType.DMA,
            pltpu.SemaphoreType.DMA,
        ],
        compiler_params=pltpu.CompilerParams(needs_layout_passes=False),
    )
    return f(pred_ins, ins_lbl)


def _disc_body(pred_ins_ref, ins_lbl_ref, seg_ref, out_ref, *, nsamp):
    i = pl.program_id(0)
    n = pred_ins_ref.shape[1]
    M = NUM_CLASSES

    pred = pred_ins_ref[0]
    lbl_row = ins_lbl_ref[0]
    seg = (seg_ref[0, 0] + seg_ref[0, 1]) + (seg_ref[0, 2] + seg_ref[0, 3])

    cls_iota = jax.lax.broadcasted_iota(jnp.int32, (M, n), 0)
    oh_t = (cls_iota == lbl_row).astype(jnp.float32)
    counts_col = jnp.sum(oh_t, axis=1, keepdims=True)

    present_col = counts_col > 0.0
    presentf_col = present_col.astype(jnp.float32)
    Kf = jnp.sum(presentf_col)
    cf_safe = jnp.where(present_col, counts_col, 1.0)
    mu = jnp.where(present_col, seg / cf_safe, 0.0)

    ones_col = jnp.ones((128, 1), dtype=jnp.float32)

    mu_exp = jax.lax.dot_general(
        oh_t, mu, (((0,), (0,)), ((), ())),
        preferred_element_type=jnp.float32)
    ad = jnp.abs(pred - mu_exp)
    dist = jnp.dot(ad, ones_col, preferred_element_type=jnp.float32)
    r = jnp.square(jnp.maximum(dist - DELTA_V, 0.0))
    segr = jnp.dot(oh_t, r, preferred_element_type=jnp.float32)
    l_var = jnp.sum(segr / cf_safe) / Kf

    diff = jnp.abs(mu[:, None, :] - mu[None, :, :]).reshape(M * M, 128)
    n1 = jnp.dot(diff, ones_col,
                 preferred_element_type=jnp.float32).reshape(M, M)
    mn = jnp.square(jnp.maximum(2.0 * DELTA_D - n1, 0.0))
    row_i = jax.lax.broadcasted_iota(jnp.int32, (M, M), 0)
    col_i = jax.lax.broadcasted_iota(jnp.int32, (M, M), 1)
    off_diag = (row_i != col_i).astype(jnp.float32)
    pair_mask = presentf_col * presentf_col.reshape(1, M) * off_diag
    denom = jnp.where(Kf > 1.0, Kf * (Kf - 1.0), 1.0)
    l_dist = jnp.where(Kf > 1.0, jnp.sum(mn * pair_mask) / denom, 0.0)

    l_reg = jnp.sum(jnp.abs(mu)) / Kf
    disc = P_VAR * l_var + P_DIST * l_dist + P_REG * l_reg

    contrib = disc / nsamp
    prev = jnp.where(i == 0, jnp.zeros((1, 1), jnp.float32), out_ref[...])
    out_ref[...] = prev + contrib


def _ce_body(pred_sem_ref, sem_lbl_ref, out_ref):
    i = pl.program_id(0)
    n = pred_sem_ref.shape[2]

    x = pred_sem_ref[0]
    sem_row = sem_lbl_ref[0]
    m = jnp.max(x, axis=0, keepdims=True)
    lse = jnp.log(jnp.sum(jnp.exp(x - m), axis=0, keepdims=True)) + m
    sem_iota = jax.lax.broadcasted_iota(jnp.int32, (NUM_SEM, n), 0)
    oh_sem_t = (sem_iota == sem_row).astype(jnp.float32)
    xl = jnp.sum(x * oh_sem_t, axis=0, keepdims=True)
    nll_sum = jnp.sum(lse - xl)

    B = pl.num_programs(0)
    contrib = nll_sum / (B * n)
    prev = jnp.where(i == 0, jnp.zeros((1, 1), jnp.float32), out_ref[...])
    out_ref[...] = prev + contrib


def _disc_call(pred_bf, ins_lbl, seg_parts, s_base, nsamp, nwave):
    n, D = pred_bf.shape[1], pred_bf.shape[2]
    return pl.pallas_call(
        functools.partial(_disc_body, nsamp=nsamp),
        grid=(nwave,),
        in_specs=[
            pl.BlockSpec((1, n, D), lambda i: (i + s_base, 0, 0)),
            pl.BlockSpec((1, 1, n), lambda i: (i + s_base, 0, 0)),
            pl.BlockSpec((1, _TPS, NUM_CLASSES, 128), lambda i: (i, 0, 0, 0)),
        ],
        out_specs=pl.BlockSpec((1, 1), lambda i: (0, 0)),
        out_shape=jax.ShapeDtypeStruct((1, 1), jnp.float32),
    )(pred_bf, ins_lbl, seg_parts)


def kernel(pred_ins, pred_sem, true_ins, true_sem):
    B, n, D = pred_ins.shape
    nwave = _NW // _TPS
    sem_t = jnp.transpose(pred_sem, (0, 2, 1))
    ins_lbl = true_ins.reshape(B, 1, n)
    sem_lbl = true_sem.reshape(B, 1, n)

    seg_a = _sc_segment_sums(pred_ins, ins_lbl, 0)
    seg_b = _sc_segment_sums(pred_ins, ins_lbl, nwave)

    ce = pl.pallas_call(
        _ce_body,
        grid=(B,),
        in_specs=[
            pl.BlockSpec((1, NUM_SEM, n), lambda i: (i, 0, 0)),
            pl.BlockSpec((1, 1, n), lambda i: (i, 0, 0)),
        ],
        out_specs=pl.BlockSpec((1, 1), lambda i: (0, 0)),
        out_shape=jax.ShapeDtypeStruct((1, 1), jnp.float32),
    )(sem_t, sem_lbl)

    disc_a = _disc_call(pred_ins, ins_lbl, seg_a, 0, B, nwave)
    disc_b = _disc_call(pred_ins, ins_lbl, seg_b, nwave, B, nwave)

    return (pred_sem, ce[0, 0] + disc_a[0, 0] + disc_b[0, 0])


# speedup vs baseline: 1.0652x; 1.0056x over previous
import functools

import jax
import jax.numpy as jnp
from jax import lax
from jax.experimental import pallas as pl
from jax.experimental.pallas import tpu as pltpu
from jax.experimental.pallas import tpu_sc as plsc

DELTA_V = 0.5
DELTA_D = 1.5
P_VAR = 1.0
P_DIST = 1.0
P_REG = 0.001
NUM_CLASSES = 24
NUM_SEM = 13

_NW = 32
_TPS = 4
_QTR = 1024
_CH = 256


def _sc_seg_body(pred_hbm, lbl_hbm, out_hbm, lbl_v, buf0, buf1, seg_v, sem0,
                 sem1, *, s_base):
    wid = lax.axis_index("s") * 2 + lax.axis_index("c")
    s_loc = wid // _TPS
    h = wid % _TPS
    s = s_base + s_loc
    base_pt = h * _QTR

    zero = jnp.zeros((16,), jnp.float32)
    for row in range(NUM_CLASSES):
        for k in range(8):
            seg_v[row, pl.ds(k * 16, 16)] = zero

    pltpu.sync_copy(lbl_hbm.at[s, pl.ds(base_pt, _QTR)], lbl_v)

    lane = lax.iota(jnp.int32, 16)
    bufs = (buf0, buf1)
    sems = (sem0, sem1)
    nch = _QTR // _CH

    def start(c):
        return pltpu.async_copy(
            pred_hbm.at[s, pl.ds(base_pt + c * _CH, _CH), :],
            bufs[c % 2], sems[c % 2])

    copies = {0: start(0)}
    for c in range(nch):
        if c + 1 < nch:
            copies[c + 1] = start(c + 1)
        copies[c].wait()
        buf = bufs[c % 2]

        def blk_body(blk, cc, c=c, buf=buf):
            p0 = blk * 16
            lbl_vec = lbl_v[pl.ds(c * _CH + p0, 16)]

            def load(j):
                vs = [buf[p0 + j, pl.ds(k * 16, 16)] for k in range(8)]
                row = jnp.full((16,), lbl_vec[j], jnp.int32)
                return vs, row

            vecs, row = load(0)
            for j in range(1, 16):
                nvecs, nrow = load(j)
                for k in range(8):
                    plsc.addupdate_scatter(
                        seg_v, [row, lane + k * 16], vecs[k])
                vecs, row = nvecs, nrow
            for k in range(8):
                plsc.addupdate_scatter(seg_v, [row, lane + k * 16], vecs[k])
            return cc

        lax.fori_loop(0, _CH // 16, blk_body, 0)

    pltpu.sync_copy(seg_v, out_hbm.at[s_loc, h])


def _sc_segment_sums(pred_ins, true_ins, s_base):
    B, n, D = pred_ins.shape
    nw_samples = _NW // _TPS
    mesh = plsc.VectorSubcoreMesh(core_axis_name="c", subcore_axis_name="s")
    f = pl.kernel(
        functools.partial(_sc_seg_body, s_base=s_base),
        out_type=jax.ShapeDtypeStruct(
            (nw_samples, _TPS, NUM_CLASSES, 128), jnp.float32),
        mesh=mesh,
        scratch_types=[
            pltpu.VMEM((_QTR,), jnp.int32),
            pltpu.VMEM((_CH, D), jnp.float32),
            pltpu.VMEM((_CH, D), jnp.float32),
            pltpu.VMEM((NUM_CLASSES, 128), jnp.float32),
            pltpu.SemaphoreType.DMA,
            pltpu.SemaphoreType.DMA,
        ],
        compiler_params=pltpu.CompilerParams(needs_layout_passes=False),
    )
    return f(pred_ins, true_ins)


def _disc_body(pred_ins_ref, ins_lbl_ref, seg_ref, out_ref, *, nsamp):
    i = pl.program_id(0)
    n = pred_ins_ref.shape[1]
    M = NUM_CLASSES

    pred = pred_ins_ref[0]
    lbl_row = ins_lbl_ref[0]
    seg = (seg_ref[0, 0] + seg_ref[0, 1]) + (seg_ref[0, 2] + seg_ref[0, 3])

    cls_iota = jax.lax.broadcasted_iota(jnp.int32, (M, n), 0)
    oh_t = (cls_iota == lbl_row).astype(jnp.float32)
    counts_col = jnp.sum(oh_t, axis=1, keepdims=True)

    present_col = counts_col > 0.0
    presentf_col = present_col.astype(jnp.float32)
    Kf = jnp.sum(presentf_col)
    cf_safe = jnp.where(present_col, counts_col, 1.0)
    mu = jnp.where(present_col, seg / cf_safe, 0.0)

    ones_col = jnp.ones((128, 1), dtype=jnp.float32)

    mu_exp = jax.lax.dot_general(
        oh_t, mu, (((0,), (0,)), ((), ())),
        preferred_element_type=jnp.float32)
    ad = jnp.abs(pred - mu_exp)
    dist = jnp.dot(ad, ones_col, preferred_element_type=jnp.float32)
    r = jnp.square(jnp.maximum(dist - DELTA_V, 0.0))
    segr = jnp.dot(oh_t, r, preferred_element_type=jnp.float32)
    l_var = jnp.sum(segr / cf_safe) / Kf

    diff = jnp.abs(mu[:, None, :] - mu[None, :, :]).reshape(M * M, 128)
    n1 = jnp.dot(diff, ones_col,
                 preferred_element_type=jnp.float32).reshape(M, M)
    mn = jnp.square(jnp.maximum(2.0 * DELTA_D - n1, 0.0))
    row_i = jax.lax.broadcasted_iota(jnp.int32, (M, M), 0)
    col_i = jax.lax.broadcasted_iota(jnp.int32, (M, M), 1)
    off_diag = (row_i != col_i).astype(jnp.float32)
    pair_mask = presentf_col * presentf_col.reshape(1, M) * off_diag
    denom = jnp.where(Kf > 1.0, Kf * (Kf - 1.0), 1.0)
    l_dist = jnp.where(Kf > 1.0, jnp.sum(mn * pair_mask) / denom, 0.0)

    l_reg = jnp.sum(jnp.abs(mu)) / Kf
    disc = P_VAR * l_var + P_DIST * l_dist + P_REG * l_reg

    contrib = disc / nsamp
    prev = jnp.where(i == 0, jnp.zeros((1, 1), jnp.float32), out_ref[...])
    out_ref[...] = prev + contrib


def _ce_body(pred_sem_ref, sem_lbl_ref, out_ref):
    i = pl.program_id(0)
    n = pred_sem_ref.shape[2]

    x = pred_sem_ref[0]
    sem_row = sem_lbl_ref[0]
    m = jnp.max(x, axis=0, keepdims=True)
    lse = jnp.log(jnp.sum(jnp.exp(x - m), axis=0, keepdims=True)) + m
    sem_iota = jax.lax.broadcasted_iota(jnp.int32, (NUM_SEM, n), 0)
    oh_sem_t = (sem_iota == sem_row).astype(jnp.float32)
    xl = jnp.sum(x * oh_sem_t, axis=0, keepdims=True)
    nll_sum = jnp.sum(lse - xl)

    B = pl.num_programs(0)
    contrib = nll_sum / (B * n)
    prev = jnp.where(i == 0, jnp.zeros((1, 1), jnp.float32), out_ref[...])
    out_ref[...] = prev + contrib


def _disc_call(pred_bf, ins_lbl, seg_parts, s_base, nsamp, nwave):
    n, D = pred_bf.shape[1], pred_bf.shape[2]
    return pl.pallas_call(
        functools.partial(_disc_body, nsamp=nsamp),
        grid=(nwave,),
        in_specs=[
            pl.BlockSpec((1, n, D), lambda i: (i + s_base, 0, 0)),
            pl.BlockSpec((1, 1, n), lambda i: (i + s_base, 0, 0)),
            pl.BlockSpec((1, _TPS, NUM_CLASSES, 128), lambda i: (i, 0, 0, 0)),
        ],
        out_specs=pl.BlockSpec((1, 1), lambda i: (0, 0)),
        out_shape=jax.ShapeDtypeStruct((1, 1), jnp.float32),
    )(pred_bf, ins_lbl, seg_parts)


def kernel(pred_ins, pred_sem, true_ins, true_sem):
    B, n, D = pred_ins.shape
    nwave = _NW // _TPS
    sem_t = jnp.transpose(pred_sem, (0, 2, 1))
    ins_lbl = true_ins.reshape(B, 1, n)
    sem_lbl = true_sem.reshape(B, 1, n)

    seg_a = _sc_segment_sums(pred_ins, true_ins, 0)
    seg_b = _sc_segment_sums(pred_ins, true_ins, nwave)

    ce = pl.pallas_call(
        _ce_body,
        grid=(B,),
        in_specs=[
            pl.BlockSpec((1, NUM_SEM, n), lambda i: (i, 0, 0)),
            pl.BlockSpec((1, 1, n), lambda i: (i, 0, 0)),
        ],
        out_specs=pl.BlockSpec((1, 1), lambda i: (0, 0)),
        out_shape=jax.ShapeDtypeStruct((1, 1), jnp.float32),
    )(sem_t, sem_lbl)

    disc_a = _disc_call(pred_ins, ins_lbl, seg_a, 0, B, nwave)
    disc_b = _disc_call(pred_ins, ins_lbl, seg_b, nwave, B, nwave)

    return (pred_sem, ce[0, 0] + disc_a[0, 0] + disc_b[0, 0])


# speedup vs baseline: 1.0666x; 1.0013x over previous
import functools

import jax
import jax.numpy as jnp
from jax import lax
from jax.experimental import pallas as pl
from jax.experimental.pallas import tpu as pltpu
from jax.experimental.pallas import tpu_sc as plsc

DELTA_V = 0.5
DELTA_D = 1.5
P_VAR = 1.0
P_DIST = 1.0
P_REG = 0.001
NUM_CLASSES = 24
NUM_SEM = 13

_NW = 32
_TPS = 4
_QTR = 1024
_CH = 256


def _sc_seg_body(pred_hbm, lbl_hbm, out_hbm, lbl_v, buf0, buf1, seg_v, sem0,
                 sem1, *, s_base):
    wid = lax.axis_index("s") * 2 + lax.axis_index("c")
    s_loc = wid // _TPS
    h = wid % _TPS
    s = s_base + s_loc
    base_pt = h * _QTR

    zero = jnp.zeros((16,), jnp.float32)
    for row in range(NUM_CLASSES):
        for k in range(8):
            seg_v[row, pl.ds(k * 16, 16)] = zero

    pltpu.sync_copy(lbl_hbm.at[s, pl.ds(base_pt, _QTR)], lbl_v)

    lane = lax.iota(jnp.int32, 16)
    bufs = (buf0, buf1)
    sems = (sem0, sem1)
    nch = _QTR // _CH

    def start(c):
        return pltpu.async_copy(
            pred_hbm.at[s, pl.ds(base_pt + c * _CH, _CH), :],
            bufs[c % 2], sems[c % 2])

    copies = {0: start(0)}
    for c in range(nch):
        if c + 1 < nch:
            copies[c + 1] = start(c + 1)
        copies[c].wait()
        buf = bufs[c % 2]

        def blk_body(blk, cc, c=c, buf=buf):
            p0 = blk * 16
            lbl_vec = lbl_v[pl.ds(c * _CH + p0, 16)]

            def load(j):
                vs = [buf[p0 + j, pl.ds(k * 16, 16)] for k in range(8)]
                row = jnp.full((16,), lbl_vec[j], jnp.int32)
                return vs, row

            vecs, row = load(0)
            for j in range(1, 16):
                nvecs, nrow = load(j)
                for k in range(8):
                    plsc.addupdate_scatter(
                        seg_v, [row, lane + k * 16], vecs[k])
                vecs, row = nvecs, nrow
            for k in range(8):
                plsc.addupdate_scatter(seg_v, [row, lane + k * 16], vecs[k])
            return cc

        lax.fori_loop(0, _CH // 16, blk_body, 0)

    pltpu.sync_copy(seg_v, out_hbm.at[s_loc, h])


def _sc_segment_sums(pred_ins, true_ins, s_base):
    B, n, D = pred_ins.shape
    nw_samples = _NW // _TPS
    mesh = plsc.VectorSubcoreMesh(core_axis_name="c", subcore_axis_name="s")
    f = pl.kernel(
        functools.partial(_sc_seg_body, s_base=s_base),
        out_type=jax.ShapeDtypeStruct(
            (nw_samples, _TPS, NUM_CLASSES, 128), jnp.float32),
        mesh=mesh,
        scratch_types=[
            pltpu.VMEM((_QTR,), jnp.int32),
            pltpu.VMEM((_CH, D), jnp.float32),
            pltpu.VMEM((_CH, D), jnp.float32),
            pltpu.VMEM((NUM_CLASSES, 128), jnp.float32),
            pltpu.SemaphoreType.DMA,
            pltpu.SemaphoreType.DMA,
        ],
        compiler_params=pltpu.CompilerParams(needs_layout_passes=False),
    )
    return f(pred_ins, true_ins)


def _disc_body(pred_ins_ref, ins_lbl_ref, seg_ref, out_ref, *, nsamp):
    i = pl.program_id(0)
    n = pred_ins_ref.shape[1]
    M = NUM_CLASSES

    pred = pred_ins_ref[0]
    lbl_row = ins_lbl_ref[0]
    seg = (seg_ref[0, 0] + seg_ref[0, 1]) + (seg_ref[0, 2] + seg_ref[0, 3])

    cls_iota = jax.lax.broadcasted_iota(jnp.int32, (M, n), 0)
    oh_t = (cls_iota == lbl_row).astype(jnp.float32)
    counts_col = jnp.sum(oh_t, axis=1, keepdims=True)

    present_col = counts_col > 0.0
    presentf_col = present_col.astype(jnp.float32)
    Kf = jnp.sum(presentf_col)
    cf_safe = jnp.where(present_col, counts_col, 1.0)
    mu = jnp.where(present_col, seg / cf_safe, 0.0)

    ones_col = jnp.ones((128, 1), dtype=jnp.float32)

    mu_exp = jax.lax.dot_general(
        oh_t, mu, (((0,), (0,)), ((), ())),
        preferred_element_type=jnp.float32)
    ad = jnp.abs(pred - mu_exp)
    dist = jnp.dot(ad, ones_col, preferred_element_type=jnp.float32)
    r = jnp.square(jnp.maximum(dist - DELTA_V, 0.0))
    segr = jnp.dot(oh_t, r, preferred_element_type=jnp.float32)
    l_var = jnp.sum(segr / cf_safe) / Kf

    diff = jnp.abs(mu[:, None, :] - mu[None, :, :]).reshape(M * M, 128)
    n1 = jnp.dot(diff, ones_col,
                 preferred_element_type=jnp.float32).reshape(M, M)
    mn = jnp.square(jnp.maximum(2.0 * DELTA_D - n1, 0.0))
    row_i = jax.lax.broadcasted_iota(jnp.int32, (M, M), 0)
    col_i = jax.lax.broadcasted_iota(jnp.int32, (M, M), 1)
    off_diag = (row_i != col_i).astype(jnp.float32)
    pair_mask = presentf_col * presentf_col.reshape(1, M) * off_diag
    denom = jnp.where(Kf > 1.0, Kf * (Kf - 1.0), 1.0)
    l_dist = jnp.where(Kf > 1.0, jnp.sum(mn * pair_mask) / denom, 0.0)

    l_reg = jnp.sum(jnp.abs(mu)) / Kf
    disc = P_VAR * l_var + P_DIST * l_dist + P_REG * l_reg

    contrib = disc / nsamp
    prev = jnp.where(i == 0, jnp.zeros((1, 1), jnp.float32), out_ref[...])
    out_ref[...] = prev + contrib


def _ce_body(pred_sem_ref, sem_lbl_ref, out_ref):
    i = pl.program_id(0)
    n = pred_sem_ref.shape[2]

    x = pred_sem_ref[0]
    sem_row = sem_lbl_ref[0]
    m = jnp.max(x, axis=0, keepdims=True)
    lse = jnp.log(jnp.sum(jnp.exp(x - m), axis=0, keepdims=True)) + m
    sem_iota = jax.lax.broadcasted_iota(jnp.int32, (NUM_SEM, n), 0)
    oh_sem_t = (sem_iota == sem_row).astype(jnp.float32)
    xl = jnp.sum(x * oh_sem_t, axis=0, keepdims=True)
    nll_sum = jnp.sum(lse - xl)

    B = pl.num_programs(0)
    contrib = nll_sum / (B * n)
    prev = jnp.where(i == 0, jnp.zeros((1, 1), jnp.float32), out_ref[...])
    out_ref[...] = prev + contrib


def _disc_call(pred_ins, ins_lbl, seg_parts, s_base, nsamp, nwave):
    n, D = pred_ins.shape[1], pred_ins.shape[2]
    return pl.pallas_call(
        functools.partial(_disc_body, nsamp=nsamp),
        grid=(nwave,),
        in_specs=[
            pl.BlockSpec((1, n, D), lambda i: (i + s_base, 0, 0)),
            pl.BlockSpec((1, 1, n), lambda i: (i + s_base, 0, 0)),
            pl.BlockSpec((1, _TPS, NUM_CLASSES, 128), lambda i: (i, 0, 0, 0)),
        ],
        out_specs=pl.BlockSpec((1, 1), lambda i: (0, 0)),
        out_shape=jax.ShapeDtypeStruct((1, 1), jnp.float32),
    )(pred_ins, ins_lbl, seg_parts)


def kernel(pred_ins, pred_sem, true_ins, true_sem):
    B, n, D = pred_ins.shape
    nwave = _NW // _TPS
    sem_t = jnp.transpose(pred_sem, (0, 2, 1))
    ins_lbl = true_ins.reshape(B, 1, n)
    sem_lbl = true_sem.reshape(B, 1, n)

    seg_a = _sc_segment_sums(pred_ins, true_ins, 0)
    seg_b = _sc_segment_sums(pred_ins, true_ins, nwave)

    ce = pl.pallas_call(
        _ce_body,
        grid=(B,),
        in_specs=[
            pl.BlockSpec((1, NUM_SEM, n), lambda i: (i, 0, 0)),
            pl.BlockSpec((1, 1, n), lambda i: (i, 0, 0)),
        ],
        out_specs=pl.BlockSpec((1, 1), lambda i: (0, 0)),
        out_shape=jax.ShapeDtypeStruct((1, 1), jnp.float32),
    )(sem_t, sem_lbl)

    disc_a = _disc_call(pred_ins, ins_lbl, seg_a, 0, B, nwave)
    disc_b = _disc_call(pred_ins, ins_lbl, seg_b, nwave, B, nwave)

    return (pred_sem, ce[0, 0] + disc_a[0, 0] + disc_b[0, 0])
